# Initial kernel scaffold; baseline (speedup 1.0000x reference)
import functools

import jax
import jax.numpy as jnp
from jax import lax
from jax.experimental import pallas as pl
from jax.experimental.pallas import tpu as pltpu
from jax.experimental.pallas import tpu_sc as plsc

NC = 2
NS = 16
NW = NC * NS
CK = 128
LANES = 16


def _make_sc_aggregate(n_nodes: int, n_chunks: int, feat: int):
  rows_per_tile = n_nodes // NS
  zrows = rows_per_tile // 5
  mesh = plsc.VectorSubcoreMesh(
      core_axis_name="c", subcore_axis_name="s", num_cores=NC,
      num_subcores=NS)

  @functools.partial(
      pl.kernel,
      out_type=jax.ShapeDtypeStruct((NC * n_nodes, feat), jnp.float32),
      mesh=mesh,
      scratch_types=[
          pltpu.VMEM((n_chunks, CK), jnp.int32),
          pltpu.VMEM((n_chunks, CK), jnp.int32),
          pltpu.VMEM((n_chunks, CK), jnp.float32),
          pltpu.VMEM((CK, feat), jnp.float32),
          pltpu.VMEM((n_nodes // NS // 5, feat), jnp.float32),
          pltpu.VMEM_SHARED((n_nodes, feat), jnp.float32),
          pltpu.SemaphoreType.DMA,
      ],
  )
  def sc_aggregate(h_hbm, src_hbm, dst_hbm, w_hbm, out_hbm,
                   src_v, dst_v, w_v, rows_v, zero_v, agg_sh, gsem):
    cid = lax.axis_index("c")
    sid = lax.axis_index("s")
    wid = sid * NC + cid

    pltpu.sync_copy(src_hbm.at[wid], src_v)
    pltpu.sync_copy(dst_hbm.at[wid], dst_v)
    pltpu.sync_copy(w_hbm.at[wid], w_v)

    zrows = n_nodes // NS // 5
    zvec = jnp.zeros((LANES,), jnp.float32)

    def zfill(i, carry):
      for k in range(feat // LANES):
        zero_v[i, pl.ds(k * LANES, LANES)] = zvec
      return carry

    lax.fori_loop(0, zrows, zfill, 0)
    for i in range(rows_per_tile // zrows):
      pltpu.sync_copy(zero_v, agg_sh.at[pl.ds(sid * rows_per_tile
                                              + i * zrows, zrows)])
    plsc.subcore_barrier()

    def chunk_body(c, carry):
      pltpu.async_copy(h_hbm.at[src_v.at[c]], rows_v, gsem).wait()

      def scale_group(g, inner):
        base = g * LANES
        for j in range(LANES):
          e = base + j
          wj = plsc.load_gather(
              w_v, [jnp.full((LANES,), c, jnp.int32),
                    jnp.full((LANES,), e, jnp.int32)])
          for k in range(feat // LANES):
            sl = pl.ds(k * LANES, LANES)
            rows_v[e, sl] = rows_v[e, sl] * wj
        return inner

      lax.fori_loop(0, CK // LANES, scale_group, 0)

      pltpu.sync_copy(rows_v, agg_sh.at[dst_v.at[c]], add=True)
      return carry

    lax.fori_loop(0, n_chunks, chunk_body, 0)
    plsc.subcore_barrier()

    pltpu.sync_copy(
        agg_sh.at[pl.ds(sid * rows_per_tile, rows_per_tile)],
        out_hbm.at[pl.ds(cid * n_nodes + sid * rows_per_tile,
                         rows_per_tile)])

  return sc_aggregate


_BLK = 400


def _layer_body(h_ref, a0_ref, a1_ref, w_ref, b_ref, o_ref):
  x = h_ref[...] + a0_ref[...] + a1_ref[...]
  y = lax.dot_general(x, w_ref[...], (((1,), (1,)), ((), ())),
                      preferred_element_type=jnp.float32)
  o_ref[...] = jnp.maximum(y + b_ref[...], 0.0)


def _tc_layer(h, a0, a1, w, b):
  n, feat = h.shape
  grid = n // _BLK
  return pl.pallas_call(
      _layer_body,
      grid=(grid,),
      in_specs=[
          pl.BlockSpec((_BLK, feat), lambda i: (i, 0)),
          pl.BlockSpec((_BLK, feat), lambda i: (i, 0)),
          pl.BlockSpec((_BLK, feat), lambda i: (i, 0)),
          pl.BlockSpec(w.shape, lambda i: (0, 0)),
          pl.BlockSpec((1, feat), lambda i: (0, 0)),
      ],
      out_specs=pl.BlockSpec((_BLK, feat), lambda i: (i, 0)),
      out_shape=jax.ShapeDtypeStruct((n, feat), jnp.float32),
  )(h, a0, a1, w, b.reshape(1, feat))


def _readout_body(h1_ref, h2_ref, h3_ref, wr_ref, br_ref, o_ref):
  feat = h1_ref.shape[1]
  dn = (((1,), (1,)), ((), ()))
  y = lax.dot_general(jnp.maximum(h1_ref[...], 0.0), wr_ref[:, 0:feat],
                      dn, preferred_element_type=jnp.float32)
  y += lax.dot_general(jnp.maximum(h2_ref[...], 0.0),
                       wr_ref[:, feat:2 * feat], dn,
                       preferred_element_type=jnp.float32)
  y += lax.dot_general(jnp.maximum(h3_ref[...], 0.0),
                       wr_ref[:, 2 * feat:3 * feat], dn,
                       preferred_element_type=jnp.float32)
  o_ref[...] = y + br_ref[...]


def _tc_readout(h1, h2, h3, wr, br):
  n, feat = h1.shape
  grid = n // _BLK
  return pl.pallas_call(
      _readout_body,
      grid=(grid,),
      in_specs=[
          pl.BlockSpec((_BLK, feat), lambda i: (i, 0)),
          pl.BlockSpec((_BLK, feat), lambda i: (i, 0)),
          pl.BlockSpec((_BLK, feat), lambda i: (i, 0)),
          pl.BlockSpec(wr.shape, lambda i: (0, 0)),
          pl.BlockSpec((1, feat), lambda i: (0, 0)),
      ],
      out_specs=pl.BlockSpec((_BLK, feat), lambda i: (i, 0)),
      out_shape=jax.ShapeDtypeStruct((n, feat), jnp.float32),
  )(h1, h2, h3, wr, br.reshape(1, feat))


def kernel(node_embed, edge_index, edge_weight, W0, b0, W1, b1, W2, b2,
           Wr, br):
  n, feat = node_embed.shape
  e = edge_index.shape[1]
  n_chunks = -(-e // (NW * CK))
  e_pad = NW * n_chunks * CK

  src = jnp.pad(edge_index[0], (0, e_pad - e)).reshape(NW, n_chunks, CK)
  dst = jnp.pad(edge_index[1], (0, e_pad - e)).reshape(NW, n_chunks, CK)
  w = jnp.pad(edge_weight, (0, e_pad - e)).reshape(NW, n_chunks, CK)

  sc_aggregate = _make_sc_aggregate(n, n_chunks, feat)

  def gin_layer(h, wmat, bvec):
    agg = sc_aggregate(h, src, dst, w)
    return _tc_layer(h, agg[:n], agg[n:], wmat, bvec)

  h1 = gin_layer(node_embed, W0, b0)
  h2 = gin_layer(h1, W1, b1)
  h3 = gin_layer(h2, W2, b2)
  return _tc_readout(h1, h2, h3, Wr, br)


# speedup vs baseline: 3.4383x; 3.4383x over previous
import functools

import jax
import jax.numpy as jnp
from jax import lax
from jax.experimental import pallas as pl
from jax.experimental.pallas import tpu as pltpu
from jax.experimental.pallas import tpu_sc as plsc

NC = 2
NS = 16
NW = NC * NS
CK = 128
LANES = 16


ZROWS = 128


def _make_sc_aggregate(n_pad: int, n_chunks: int, feat: int):
  rows_per_tile = n_pad // NS
  mesh = plsc.VectorSubcoreMesh(
      core_axis_name="c", subcore_axis_name="s", num_cores=NC,
      num_subcores=NS)

  @functools.partial(
      pl.kernel,
      out_type=jax.ShapeDtypeStruct((NC * n_pad, feat), jnp.float32),
      mesh=mesh,
      scratch_types=[
          pltpu.VMEM((2, CK), jnp.int32),
          pltpu.VMEM((CK,), jnp.float32),
          pltpu.VMEM((CK, feat), jnp.float32),
          pltpu.VMEM_SHARED((n_pad, feat), jnp.float32),
          pltpu.SemaphoreType.DMA,
      ],
  )
  def sc_aggregate(h_hbm, eidx_hbm, ew_hbm, out_hbm, ebuf, wbuf, rows_v,
                   agg_sh, gsem):
    cid = lax.axis_index("c")
    sid = lax.axis_index("s")
    wid = sid * NC + cid

    zvec = jnp.zeros((LANES,), jnp.float32)

    def zfill(i, carry):
      for k in range(feat // LANES):
        rows_v[i, pl.ds(k * LANES, LANES)] = zvec
      return carry

    lax.fori_loop(0, ZROWS, zfill, 0)
    for i in range(rows_per_tile // ZROWS):
      pltpu.sync_copy(rows_v, agg_sh.at[pl.ds(sid * rows_per_tile
                                              + i * ZROWS, ZROWS)])
    plsc.subcore_barrier()

    def chunk_body(c, carry):
      pltpu.sync_copy(eidx_hbm.at[wid, c], ebuf)
      pltpu.sync_copy(ew_hbm.at[wid, c], wbuf)
      pltpu.async_copy(h_hbm.at[ebuf.at[0]], rows_v, gsem).wait()

      def scale_group(g, inner):
        base = g * LANES
        w16 = wbuf[pl.ds(base, LANES)]
        for j in range(LANES):
          e = base + j
          wj = lax.gather(
              w16, jnp.full((LANES, 1), j, jnp.int32),
              lax.GatherDimensionNumbers(offset_dims=(),
                                         collapsed_slice_dims=(0,),
                                         start_index_map=(0,)),
              slice_sizes=(1,),
              mode=lax.GatherScatterMode.PROMISE_IN_BOUNDS)
          for k in range(feat // LANES):
            sl = pl.ds(k * LANES, LANES)
            rows_v[e, sl] = rows_v[e, sl] * wj
        return inner

      lax.fori_loop(0, CK // LANES, scale_group, 0)

      pltpu.sync_copy(rows_v, agg_sh.at[ebuf.at[1]], add=True)
      return carry

    lax.fori_loop(0, n_chunks, chunk_body, 0)
    plsc.subcore_barrier()

    pltpu.sync_copy(
        agg_sh.at[pl.ds(sid * rows_per_tile, rows_per_tile)],
        out_hbm.at[pl.ds(cid * n_pad + sid * rows_per_tile,
                         rows_per_tile)])

  return sc_aggregate


_BLK = 400


def _layer_body(h_ref, a0_ref, a1_ref, w_ref, b_ref, o_ref):
  x = h_ref[...] + a0_ref[...] + a1_ref[...]
  y = lax.dot_general(x, w_ref[...], (((1,), (1,)), ((), ())),
                      preferred_element_type=jnp.float32)
  o_ref[...] = jnp.maximum(y + b_ref[...], 0.0)


def _tc_layer(h, a0, a1, w, b):
  n, feat = h.shape
  grid = n // _BLK
  return pl.pallas_call(
      _layer_body,
      grid=(grid,),
      in_specs=[
          pl.BlockSpec((_BLK, feat), lambda i: (i, 0)),
          pl.BlockSpec((_BLK, feat), lambda i: (i, 0)),
          pl.BlockSpec((_BLK, feat), lambda i: (i, 0)),
          pl.BlockSpec(w.shape, lambda i: (0, 0)),
          pl.BlockSpec((1, feat), lambda i: (0, 0)),
      ],
      out_specs=pl.BlockSpec((_BLK, feat), lambda i: (i, 0)),
      out_shape=jax.ShapeDtypeStruct((n, feat), jnp.float32),
  )(h, a0, a1, w, b.reshape(1, feat))


def _readout_body(h1_ref, h2_ref, h3_ref, wr_ref, br_ref, o_ref):
  feat = h1_ref.shape[1]
  dn = (((1,), (1,)), ((), ()))
  y = lax.dot_general(jnp.maximum(h1_ref[...], 0.0), wr_ref[:, 0:feat],
                      dn, preferred_element_type=jnp.float32)
  y += lax.dot_general(jnp.maximum(h2_ref[...], 0.0),
                       wr_ref[:, feat:2 * feat], dn,
                       preferred_element_type=jnp.float32)
  y += lax.dot_general(jnp.maximum(h3_ref[...], 0.0),
                       wr_ref[:, 2 * feat:3 * feat], dn,
                       preferred_element_type=jnp.float32)
  o_ref[...] = y + br_ref[...]


def _tc_readout(h1, h2, h3, wr, br):
  n, feat = h1.shape
  grid = n // _BLK
  return pl.pallas_call(
      _readout_body,
      grid=(grid,),
      in_specs=[
          pl.BlockSpec((_BLK, feat), lambda i: (i, 0)),
          pl.BlockSpec((_BLK, feat), lambda i: (i, 0)),
          pl.BlockSpec((_BLK, feat), lambda i: (i, 0)),
          pl.BlockSpec(wr.shape, lambda i: (0, 0)),
          pl.BlockSpec((1, feat), lambda i: (0, 0)),
      ],
      out_specs=pl.BlockSpec((_BLK, feat), lambda i: (i, 0)),
      out_shape=jax.ShapeDtypeStruct((n, feat), jnp.float32),
  )(h1, h2, h3, wr, br.reshape(1, feat))


def kernel(node_embed, edge_index, edge_weight, W0, b0, W1, b1, W2, b2,
           Wr, br):
  n, feat = node_embed.shape
  e = edge_index.shape[1]
  n_chunks = -(-e // (NW * CK))
  e_pad = NW * n_chunks * CK
  n_pad = -(-n // (NS * ZROWS)) * NS * ZROWS

  src = jnp.pad(edge_index[0], (0, e_pad - e)).reshape(NW, n_chunks, CK)
  dst = jnp.pad(edge_index[1], (0, e_pad - e)).reshape(NW, n_chunks, CK)
  eidx = jnp.stack([src, dst], axis=2)
  ew = jnp.pad(edge_weight, (0, e_pad - e)).reshape(NW, n_chunks, CK)

  sc_aggregate = _make_sc_aggregate(n_pad, n_chunks, feat)

  def gin_layer(h, wmat, bvec):
    agg = sc_aggregate(h, eidx, ew)
    return _tc_layer(h, agg[:n], agg[n_pad:n_pad + n], wmat, bvec)

  h1 = gin_layer(node_embed, W0, b0)
  h2 = gin_layer(h1, W1, b1)
  h3 = gin_layer(h2, W2, b2)
  return _tc_readout(h1, h2, h3, Wr, br)
